# baseline (speedup 1.0000x reference)
import functools

import jax
import jax.numpy as jnp
from jax import lax
from jax.experimental import pallas as pl
from jax.experimental.pallas import tpu as pltpu
from jax.experimental.pallas import tpu_sc as plsc

_B, _N, _C, _H, _D = 8, 1025, 3, 16, 64
_NJ = 4
_CH = _N // _NJ
_W = 8
_NIT = _CH // _W

_mesh = plsc.VectorSubcoreMesh(core_axis_name="c", subcore_axis_name="s")


@functools.partial(
    pl.kernel,
    mesh=_mesh,
    out_type=jax.ShapeDtypeStruct((_B, _N, _C, _H, _D), jnp.float32),
    scratch_types=[
        pltpu.VMEM((_W, _C, _H, _D), jnp.float32),
        pltpu.VMEM((_W, _C, _H, _D), jnp.float32),
        pltpu.SemaphoreType.DMA,
        pltpu.SemaphoreType.DMA,
        pltpu.SemaphoreType.DMA,
        pltpu.SemaphoreType.DMA,
    ],
    compiler_params=pltpu.CompilerParams(use_tc_tiling_on_sc=False),
)
def _sc_copy(x_hbm, o_hbm, buf0, buf1, si0, si1, so0, so1):
    nc = 2
    wid = lax.axis_index("s") * nc + lax.axis_index("c")
    b = wid // _NJ
    j = wid % _NJ
    base = j * _CH
    bufs = (buf0, buf1)
    sins = (si0, si1)
    souts = (so0, so1)

    def src(t):
        return x_hbm.at[b, pl.ds(base + t * _W, _W)]

    def dst(t):
        return o_hbm.at[b, pl.ds(base + t * _W, _W)]

    _UNROLL = 8
    _NG = _NIT // _UNROLL

    pltpu.async_copy(src(0), bufs[0], sins[0])

    def body(g, carry):
        for u in range(_UNROLL):
            t = g * _UNROLL + u
            k = u % 2
            pltpu.make_async_copy(src(t), bufs[k], sins[k]).wait()

            def _wait_prev(kk=1 - k, tt=t):
                pltpu.make_async_copy(
                    bufs[kk], dst(tt - 1), souts[kk]
                ).wait()

            if u == 0:
                pl.when(g > 0)(_wait_prev)
            else:
                _wait_prev()
            pltpu.async_copy(bufs[k], dst(t), souts[k])

            def _next_in(kk=1 - k, tt=t):
                pltpu.async_copy(src(tt + 1), bufs[kk], sins[kk])

            if u == _UNROLL - 1:
                pl.when(g + 1 < _NG)(_next_in)
            else:
                _next_in()
        return carry

    lax.fori_loop(0, _NG, body, 0)
    k_last = (_NIT - 1) % 2
    pltpu.make_async_copy(
        bufs[k_last], dst(_NIT - 1), souts[k_last]
    ).wait()

    @pl.when(j == 0)
    def _():
        rest = _N - _NJ * _CH
        pltpu.sync_copy(
            x_hbm.at[b, pl.ds(_NJ * _CH, rest)], buf0.at[pl.ds(0, rest)]
        )
        pltpu.sync_copy(
            buf0.at[pl.ds(0, rest)], o_hbm.at[b, pl.ds(_NJ * _CH, rest)]
        )


def kernel(x):
    return _sc_copy(x)


# speedup vs baseline: 1.2506x; 1.2506x over previous
import functools

import jax
import jax.numpy as jnp
from jax import lax
from jax.experimental import pallas as pl
from jax.experimental.pallas import tpu as pltpu
from jax.experimental.pallas import tpu_sc as plsc

_B, _N, _C, _H, _D = 8, 1025, 3, 16, 64
_NJ = 4
_CH = _N // _NJ
_W = 8
_NIT = _CH // _W

_mesh = plsc.VectorSubcoreMesh(core_axis_name="c", subcore_axis_name="s")


@functools.partial(
    pl.kernel,
    mesh=_mesh,
    out_type=jax.ShapeDtypeStruct((_B, _N, _C, _H, _D), jnp.float32),
    scratch_types=[
        pltpu.VMEM((_W, _C, _H, _D), jnp.float32),
        pltpu.VMEM((_W, _C, _H, _D), jnp.float32),
        pltpu.SemaphoreType.DMA,
        pltpu.SemaphoreType.DMA,
        pltpu.SemaphoreType.DMA,
        pltpu.SemaphoreType.DMA,
    ],
)
def _sc_copy(x_hbm, o_hbm, buf0, buf1, si0, si1, so0, so1):
    nc = 2
    wid = lax.axis_index("s") * nc + lax.axis_index("c")
    b = wid // _NJ
    j = wid % _NJ
    base = j * _CH
    bufs = (buf0, buf1)
    sins = (si0, si1)
    souts = (so0, so1)

    def src(i):
        return x_hbm.at[b, pl.ds(base + i * _W, _W)]

    def dst(i):
        return o_hbm.at[b, pl.ds(base + i * _W, _W)]

    pltpu.make_async_copy(src(0), bufs[0], sins[0]).start()
    for i in range(_NIT):
        k = i % 2
        pltpu.make_async_copy(src(i), bufs[k], sins[k]).wait()
        if i > 0:
            pltpu.make_async_copy(bufs[1 - k], dst(i - 1), souts[1 - k]).wait()
        pltpu.make_async_copy(bufs[k], dst(i), souts[k]).start()
        if i + 1 < _NIT:
            pltpu.make_async_copy(src(i + 1), bufs[1 - k], sins[1 - k]).start()
    k_last = (_NIT - 1) % 2
    pltpu.make_async_copy(bufs[k_last], dst(_NIT - 1), souts[k_last]).wait()

    @pl.when(j == 0)
    def _():
        rest = _N - _NJ * _CH
        pltpu.sync_copy(
            x_hbm.at[b, pl.ds(_NJ * _CH, rest)], buf0.at[pl.ds(0, rest)]
        )
        pltpu.sync_copy(
            buf0.at[pl.ds(0, rest)], o_hbm.at[b, pl.ds(_NJ * _CH, rest)]
        )


def kernel(x):
    return _sc_copy(x)
